# baseline (speedup 1.0000x reference)
import functools

import jax
import jax.numpy as jnp
from jax import lax
from jax.experimental import pallas as pl
from jax.experimental.pallas import tpu as pltpu
from jax.experimental.pallas import tpu_sc as plsc

_N_E = 8192
_E_DIM = 256
_BETA = 0.25
_BM = 512
_N_TOK = 8192
_GRID = _N_TOK // _BM
_N_ELEM = _N_TOK * _E_DIM


def _vq_tc_kernel(z_ref, e_ref, cols_ref, d_ref, idx_ref, loss_ref, en_ref):
    i = pl.program_id(0)

    @pl.when(i == 0)
    def _init():
        loss_ref[...] = jnp.zeros_like(loss_ref)
        for c in range(8):
            sl = pl.ds(c * 1024, 1024)
            chunk = e_ref[sl, :]
            en_ref[:, sl] = jnp.sum(chunk * chunk, axis=1, keepdims=True).T

    z = z_ref[0].T
    zn = jnp.sum(z * z, axis=1, keepdims=True)
    cross = lax.dot_general(z, e_ref[...], (((1,), (1,)), ((), ())),
                            preferred_element_type=jnp.float32)
    d = (zn + en_ref[...]) - 2.0 * cross
    d_ref[...] = d
    m = jnp.min(d, axis=1, keepdims=True)
    idxf = jnp.min(jnp.where(d == m, cols_ref[...], float(_N_E)), axis=1)
    idx_ref[0, 0, :] = idxf.astype(jnp.int32)
    loss_ref[...] += jnp.sum(m, axis=0, keepdims=True)

    @pl.when(i == _GRID - 1)
    def _fin():
        loss_ref[...] = loss_ref[...] * ((1.0 + _BETA) / _N_ELEM)


def _tc_part(z3, e):
    cols = jnp.arange(_N_E, dtype=jnp.float32).reshape(1, _N_E)
    return pl.pallas_call(
        _vq_tc_kernel,
        grid=(_GRID,),
        in_specs=[
            pl.BlockSpec((1, _E_DIM, _BM), lambda i: (i // 2, 0, i % 2)),
            pl.BlockSpec((_N_E, _E_DIM), lambda i: (0, 0)),
            pl.BlockSpec((1, _N_E), lambda i: (0, 0)),
        ],
        out_specs=[
            pl.BlockSpec((_BM, _N_E), lambda i: (i, 0)),
            pl.BlockSpec((1, 1, _BM), lambda i: (i, 0, 0)),
            pl.BlockSpec((1, 1), lambda i: (0, 0)),
        ],
        out_shape=[
            jax.ShapeDtypeStruct((_N_TOK, _N_E), jnp.float32),
            jax.ShapeDtypeStruct((_GRID, 1, _BM), jnp.int32),
            jax.ShapeDtypeStruct((1, 1), jnp.float32),
        ],
        scratch_shapes=[pltpu.VMEM((1, _N_E), jnp.float32)],
    )(z3, e, cols)


_NW = 32
_BPW = _N_TOK // _NW


def _sc_gather(table, idx):
    mesh = plsc.VectorSubcoreMesh(core_axis_name="c", subcore_axis_name="s")

    @functools.partial(
        pl.kernel,
        mesh=mesh,
        out_type=jax.ShapeDtypeStruct((_N_TOK, _E_DIM), jnp.float32),
        scratch_types=[
            pltpu.VMEM((_BPW,), jnp.int32),
            pltpu.VMEM((_BPW, _E_DIM), jnp.float32),
            pltpu.SemaphoreType.DMA,
        ],
    )
    def gather(table_hbm, idx_hbm, out_hbm, idx_v, rows_v, sem):
        wid = lax.axis_index("s") * 2 + lax.axis_index("c")
        base = wid * _BPW
        pltpu.sync_copy(idx_hbm.at[pl.ds(base, _BPW)], idx_v)
        pltpu.async_copy(table_hbm.at[idx_v], rows_v, sem).wait()
        pltpu.sync_copy(rows_v, out_hbm.at[pl.ds(base, _BPW)])

    return gather(table, idx)


def kernel(z, embedding_weight):
    z3 = z.reshape(8, _E_DIM, 1024)
    d, idx3, loss11 = _tc_part(z3, embedding_weight)
    min_encoding_indices = idx3.reshape(_N_TOK)
    z_q_flat = _sc_gather(embedding_weight, min_encoding_indices)
    z_q = z_q_flat.reshape(8, 32, 32, _E_DIM).transpose(0, 3, 1, 2)
    loss = loss11.reshape(())
    return (z_q, loss, d, min_encoding_indices)


# speedup vs baseline: 1.0869x; 1.0869x over previous
import functools

import jax
import jax.numpy as jnp
from jax import lax
from jax.experimental import pallas as pl
from jax.experimental.pallas import tpu as pltpu
from jax.experimental.pallas import tpu_sc as plsc

_N_E = 8192
_E_DIM = 256
_BETA = 0.25
_BM = 512
_N_TOK = 8192
_GRID = _N_TOK // _BM
_N_ELEM = _N_TOK * _E_DIM


def _vq_tc_kernel(z_ref, e_ref, cols_ref, d_ref, idx_ref, loss_ref, en_ref):
    i = pl.program_id(0)

    @pl.when(i == 0)
    def _init():
        loss_ref[...] = jnp.zeros_like(loss_ref)
        for c in range(8):
            sl = pl.ds(c * 1024, 1024)
            chunk = e_ref[sl, :]
            en_ref[:, sl] = jnp.sum(chunk * chunk, axis=1, keepdims=True).T

    z = z_ref[...]
    zn = jnp.sum(z * z, axis=1, keepdims=True)
    cross = lax.dot_general(z, e_ref[...], (((1,), (1,)), ((), ())),
                            preferred_element_type=jnp.float32)
    d = (zn + en_ref[...]) - 2.0 * cross
    d_ref[...] = d
    m = jnp.min(d, axis=1, keepdims=True)
    idxf = jnp.min(jnp.where(d == m, cols_ref[...], float(_N_E)), axis=1)
    idx_ref[0, 0, :] = idxf.astype(jnp.int32)
    loss_ref[...] += jnp.sum(m, axis=0, keepdims=True)

    @pl.when(i == _GRID - 1)
    def _fin():
        loss_ref[...] = loss_ref[...] * ((1.0 + _BETA) / _N_ELEM)


def _tc_part(z_flat, e):
    cols = jnp.arange(_N_E, dtype=jnp.float32).reshape(1, _N_E)
    return pl.pallas_call(
        _vq_tc_kernel,
        grid=(_GRID,),
        in_specs=[
            pl.BlockSpec((_BM, _E_DIM), lambda i: (i, 0)),
            pl.BlockSpec((_N_E, _E_DIM), lambda i: (0, 0)),
            pl.BlockSpec((1, _N_E), lambda i: (0, 0)),
        ],
        out_specs=[
            pl.BlockSpec((_BM, _N_E), lambda i: (i, 0)),
            pl.BlockSpec((1, 1, _BM), lambda i: (i, 0, 0)),
            pl.BlockSpec((1, 1), lambda i: (0, 0)),
        ],
        out_shape=[
            jax.ShapeDtypeStruct((_N_TOK, _N_E), jnp.float32),
            jax.ShapeDtypeStruct((_GRID, 1, _BM), jnp.int32),
            jax.ShapeDtypeStruct((1, 1), jnp.float32),
        ],
        scratch_shapes=[pltpu.VMEM((1, _N_E), jnp.float32)],
    )(z_flat, e, cols)


_NW = 32
_BPW = _N_TOK // _NW


def _sc_gather(table, idx):
    mesh = plsc.VectorSubcoreMesh(core_axis_name="c", subcore_axis_name="s")

    @functools.partial(
        pl.kernel,
        mesh=mesh,
        out_type=jax.ShapeDtypeStruct((_N_TOK, _E_DIM), jnp.float32),
        scratch_types=[
            pltpu.VMEM((_BPW,), jnp.int32),
            pltpu.VMEM((_BPW, _E_DIM), jnp.float32),
            pltpu.SemaphoreType.DMA,
        ],
    )
    def gather(table_hbm, idx_hbm, out_hbm, idx_v, rows_v, sem):
        wid = lax.axis_index("s") * 2 + lax.axis_index("c")
        base = wid * _BPW
        pltpu.sync_copy(idx_hbm.at[pl.ds(base, _BPW)], idx_v)
        pltpu.async_copy(table_hbm.at[idx_v], rows_v, sem).wait()
        pltpu.sync_copy(rows_v, out_hbm.at[pl.ds(base, _BPW)])

    return gather(table, idx)


def kernel(z, embedding_weight):
    zt = jnp.transpose(z, (0, 2, 3, 1))
    z_flat = zt.reshape(_N_TOK, _E_DIM)
    d, idx3, loss11 = _tc_part(z_flat, embedding_weight)
    min_encoding_indices = idx3.reshape(_N_TOK)
    z_q_flat = _sc_gather(embedding_weight, min_encoding_indices)
    z_q = z_q_flat.reshape(8, 32, 32, _E_DIM).transpose(0, 3, 1, 2)
    loss = loss11.reshape(())
    return (z_q, loss, d, min_encoding_indices)
